# baseline (speedup 1.0000x reference)
import functools

import numpy as np
import jax
import jax.numpy as jnp
from jax import lax
from jax.experimental import pallas as pl
from jax.experimental.pallas import tpu as pltpu
from jax.experimental.pallas import tpu_sc as plsc

D = 64
S = 200
B = 4096
NS, L = 16, 16
HALF = B // 2
SEQ_PER_W = HALF // NS
C = 2
CHUNK_ROWS = C * S
N_CHUNK = SEQ_PER_W // C


def _pe_table():
    position = jnp.arange(S, dtype=jnp.float32)[:, None]
    div_term = jnp.exp(
        jnp.arange(0, D, 2, dtype=jnp.float32) * (-np.log(10000.0) / D)
    )
    pe = jnp.zeros((S, D), jnp.float32)
    pe = pe.at[:, 0::2].set(jnp.sin(position * div_term))
    pe = pe.at[:, 1::2].set(jnp.cos(position * div_term))
    return pe


def _make_half(core_axis, subcore_axis):
    mesh = plsc.VectorSubcoreMesh(
        core_axis_name=core_axis, subcore_axis_name=subcore_axis, num_cores=1
    )

    @functools.partial(
        pl.kernel,
        out_type=jax.ShapeDtypeStruct((HALF * S, D), jnp.float32),
        mesh=mesh,
        compiler_params=pltpu.CompilerParams(use_tc_tiling_on_sc=False),
        scratch_types=[
            pltpu.VMEM((CHUNK_ROWS,), jnp.int32),
            pltpu.VMEM((CHUNK_ROWS, D), jnp.float32),
            pltpu.VMEM((S, D), jnp.float32),
            pltpu.SemaphoreType.DMA,
        ],
    )
    def _half(x_hbm, table_hbm, pe_hbm, out_hbm, idx_v, rows_v, pe_v, sem):
        wid = lax.axis_index(subcore_axis)
        pltpu.sync_copy(pe_hbm, pe_v)
        base_row = wid * SEQ_PER_W * S

        def chunk_body(it, carry):
            row0 = base_row + it * CHUNK_ROWS
            pltpu.sync_copy(x_hbm.at[pl.ds(row0, CHUNK_ROWS)], idx_v)
            pltpu.async_copy(table_hbm.at[idx_v], rows_v, sem).wait()

            def add_body(r, c2):
                for j in range(D // L):
                    pe_j = pe_v[r, pl.ds(j * L, L)]
                    for c in range(C):
                        rr = c * S + r
                        rows_v[rr, pl.ds(j * L, L)] = (
                            rows_v[rr, pl.ds(j * L, L)] + pe_j
                        )
                return c2

            lax.fori_loop(0, S, add_body, 0)
            pltpu.sync_copy(rows_v, out_hbm.at[pl.ds(row0, CHUNK_ROWS)])
            return carry

        lax.fori_loop(0, N_CHUNK, chunk_body, 0)

    return _half


_half0 = _make_half("c0", "s0")
_half1 = _make_half("c1", "s1")


def kernel(x, table):
    pe = _pe_table()
    xf = x.reshape(-1)
    lo = _half0(xf[: HALF * S], table, pe)
    hi = _half1(xf[HALF * S :], table, pe)
    out = jnp.concatenate(
        [lo.reshape(HALF, S, D), hi.reshape(HALF, S, D)], axis=0
    )
    return out


# speedup vs baseline: 1.3074x; 1.3074x over previous
import functools

import numpy as np
import jax
import jax.numpy as jnp
from jax import lax
from jax.experimental import pallas as pl
from jax.experimental.pallas import tpu as pltpu
from jax.experimental.pallas import tpu_sc as plsc

D = 64
S = 200
B = 4096
NC, NS, L = 2, 16, 16
NW = NC * NS
SEQ_PER_W = B // NW
C = 2
CHUNK_ROWS = C * S
N_CHUNK = SEQ_PER_W // C


def _pe_table():
    position = jnp.arange(S, dtype=jnp.float32)[:, None]
    div_term = jnp.exp(
        jnp.arange(0, D, 2, dtype=jnp.float32) * (-np.log(10000.0) / D)
    )
    pe = jnp.zeros((S, D), jnp.float32)
    pe = pe.at[:, 0::2].set(jnp.sin(position * div_term))
    pe = pe.at[:, 1::2].set(jnp.cos(position * div_term))
    return pe


_mesh = plsc.VectorSubcoreMesh(core_axis_name="c", subcore_axis_name="s")


@functools.partial(
    pl.kernel,
    out_type=jax.ShapeDtypeStruct((B * S, D), jnp.float32),
    mesh=_mesh,
    compiler_params=pltpu.CompilerParams(use_tc_tiling_on_sc=False),
    scratch_types=[
        pltpu.VMEM((N_CHUNK * CHUNK_ROWS,), jnp.int32),
        pltpu.VMEM((CHUNK_ROWS, D), jnp.float32),
        pltpu.VMEM((CHUNK_ROWS, D), jnp.float32),
        pltpu.VMEM((S, D), jnp.float32),
        pltpu.SemaphoreType.DMA,
        pltpu.SemaphoreType.DMA,
        pltpu.SemaphoreType.DMA,
        pltpu.SemaphoreType.DMA,
    ],
)
def _emb_kernel(
    x_hbm, table_hbm, pe_hbm, out_hbm,
    idx_v, rbuf0, rbuf1, pe_v,
    rsem0, rsem1, wsem0, wsem1,
):
    wid = lax.axis_index("s") * NC + lax.axis_index("c")
    base_row = wid * SEQ_PER_W * S
    pltpu.sync_copy(pe_hbm, pe_v)
    pltpu.sync_copy(x_hbm.at[pl.ds(base_row, SEQ_PER_W * S)], idx_v)

    def add_pe(rbuf):
        def add_body(r, carry):
            for j in range(D // L):
                pe_j = pe_v[r, pl.ds(j * L, L)]
                for c in range(C):
                    rr = c * S + r
                    rbuf[rr, pl.ds(j * L, L)] = rbuf[rr, pl.ds(j * L, L)] + pe_j
            return carry

        lax.fori_loop(0, S, add_body, 0)

    def gather(k, rbuf, rsem):
        pltpu.async_copy(
            table_hbm.at[idx_v.at[pl.ds(k * CHUNK_ROWS, CHUNK_ROWS)]], rbuf, rsem
        )

    def wr(k, rbuf, wsem):
        pltpu.async_copy(
            rbuf, out_hbm.at[pl.ds(base_row + k * CHUNK_ROWS, CHUNK_ROWS)], wsem
        )

    def wr_wait(k, rbuf, wsem):
        pltpu.make_async_copy(
            rbuf, out_hbm.at[pl.ds(base_row + k * CHUNK_ROWS, CHUNK_ROWS)], wsem
        ).wait()

    gather(0, rbuf0, rsem0)

    def body2(g2, carry):
        k0 = 2 * g2
        k1 = k0 + 1
        pltpu.make_async_copy(
            table_hbm.at[idx_v.at[pl.ds(k0 * CHUNK_ROWS, CHUNK_ROWS)]], rbuf0, rsem0
        ).wait()

        @pl.when(g2 > 0)
        def _():
            wr_wait(k0 - 1, rbuf1, wsem1)

        gather(k1, rbuf1, rsem1)
        add_pe(rbuf0)
        wr(k0, rbuf0, wsem0)

        pltpu.make_async_copy(
            table_hbm.at[idx_v.at[pl.ds(k1 * CHUNK_ROWS, CHUNK_ROWS)]], rbuf1, rsem1
        ).wait()
        wr_wait(k0, rbuf0, wsem0)

        @pl.when(g2 < N_CHUNK // 2 - 1)
        def _():
            gather(k1 + 1, rbuf0, rsem0)

        add_pe(rbuf1)
        wr(k1, rbuf1, wsem1)
        return carry

    lax.fori_loop(0, N_CHUNK // 2, body2, 0)
    wr_wait(N_CHUNK - 1, rbuf1, wsem1)


def kernel(x, table):
    pe = _pe_table()
    out = _emb_kernel(x.reshape(-1), table, pe)
    return out.reshape(B, S, D)


# speedup vs baseline: 1.3103x; 1.0023x over previous
import functools

import numpy as np
import jax
import jax.numpy as jnp
from jax import lax
from jax.experimental import pallas as pl
from jax.experimental.pallas import tpu as pltpu
from jax.experimental.pallas import tpu_sc as plsc

D = 64
S = 200
B = 4096
NC, NS, L = 2, 16, 16
NW = NC * NS
SEQ_PER_W = B // NW
C = 2
CHUNK_ROWS = C * S
N_CHUNK = SEQ_PER_W // C


def _pe_table():
    position = np.arange(S, dtype=np.float32)[:, None]
    div_term = np.exp(
        np.arange(0, D, 2, dtype=np.float32) * (-np.log(10000.0) / D)
    )
    pe = np.zeros((S, D), np.float32)
    pe[:, 0::2] = np.sin(position * div_term)
    pe[:, 1::2] = np.cos(position * div_term)
    return jnp.asarray(pe)


_mesh = plsc.VectorSubcoreMesh(core_axis_name="c", subcore_axis_name="s")


@functools.partial(
    pl.kernel,
    out_type=jax.ShapeDtypeStruct((B * S, D), jnp.float32),
    mesh=_mesh,
    compiler_params=pltpu.CompilerParams(use_tc_tiling_on_sc=False),
    cost_estimate=pl.CostEstimate(
        flops=B * S * D, bytes_accessed=440_000_000, transcendentals=0
    ),
    scratch_types=[
        pltpu.VMEM((N_CHUNK * CHUNK_ROWS,), jnp.int32),
        pltpu.VMEM((CHUNK_ROWS, D), jnp.float32),
        pltpu.VMEM((CHUNK_ROWS, D), jnp.float32),
        pltpu.VMEM((S, D), jnp.float32),
        pltpu.SemaphoreType.DMA,
        pltpu.SemaphoreType.DMA,
        pltpu.SemaphoreType.DMA,
        pltpu.SemaphoreType.DMA,
    ],
)
def _emb_kernel(
    x_hbm, table_hbm, pe_hbm, out_hbm,
    idx_v, rbuf0, rbuf1, pe_v,
    rsem0, rsem1, wsem0, wsem1,
):
    wid = lax.axis_index("s") * NC + lax.axis_index("c")
    base_row = wid * SEQ_PER_W * S
    pltpu.sync_copy(pe_hbm, pe_v)
    pltpu.sync_copy(x_hbm.at[pl.ds(base_row, SEQ_PER_W * S)], idx_v)

    def add_pe(rbuf):
        def add_body(r, carry):
            for j in range(D // L):
                pe_j = pe_v[r, pl.ds(j * L, L)]
                for c in range(C):
                    rr = c * S + r
                    rbuf[rr, pl.ds(j * L, L)] = rbuf[rr, pl.ds(j * L, L)] + pe_j
            return carry

        lax.fori_loop(0, S, add_body, 0)

    def gather(k, rbuf, rsem):
        pltpu.async_copy(
            table_hbm.at[idx_v.at[pl.ds(k * CHUNK_ROWS, CHUNK_ROWS)]], rbuf, rsem
        )

    def wr(k, rbuf, wsem):
        pltpu.async_copy(
            rbuf, out_hbm.at[pl.ds(base_row + k * CHUNK_ROWS, CHUNK_ROWS)], wsem
        )

    def wr_wait(k, rbuf, wsem):
        pltpu.make_async_copy(
            rbuf, out_hbm.at[pl.ds(base_row + k * CHUNK_ROWS, CHUNK_ROWS)], wsem
        ).wait()

    gather(0, rbuf0, rsem0)

    def body2(g2, carry):
        k0 = 2 * g2
        k1 = k0 + 1
        pltpu.make_async_copy(
            table_hbm.at[idx_v.at[pl.ds(k0 * CHUNK_ROWS, CHUNK_ROWS)]], rbuf0, rsem0
        ).wait()

        @pl.when(g2 > 0)
        def _():
            wr_wait(k0 - 1, rbuf1, wsem1)

        gather(k1, rbuf1, rsem1)
        add_pe(rbuf0)
        wr(k0, rbuf0, wsem0)

        pltpu.make_async_copy(
            table_hbm.at[idx_v.at[pl.ds(k1 * CHUNK_ROWS, CHUNK_ROWS)]], rbuf1, rsem1
        ).wait()
        wr_wait(k0, rbuf0, wsem0)

        @pl.when(g2 < N_CHUNK // 2 - 1)
        def _():
            gather(k1 + 1, rbuf0, rsem0)

        add_pe(rbuf1)
        wr(k1, rbuf1, wsem1)
        return carry

    lax.fori_loop(0, N_CHUNK // 2, body2, 0)
    wr_wait(N_CHUNK - 1, rbuf1, wsem1)


def kernel(x, table):
    pe = _pe_table()
    out = _emb_kernel(x.reshape(-1), table, pe)
    return out.reshape(B, S, D)
